# baseline (speedup 1.0000x reference)
import functools

import jax
import jax.numpy as jnp
from jax import lax
from jax.experimental import pallas as pl
from jax.experimental.pallas import tpu as pltpu
from jax.experimental.pallas import tpu_sc as plsc

F32 = jnp.float32

_SC_NC = 2
_SC_NS = 16
_SC_NW = _SC_NC * _SC_NS


def _sc_gather_body(table_hbm, idx_hbm, out_hbm, idx_v, rows_v, sem):
    wid = lax.axis_index("s") * _SC_NC + lax.axis_index("c")
    base = wid * 16
    pltpu.sync_copy(idx_hbm.at[pl.ds(base, 16)], idx_v)
    pltpu.async_copy(table_hbm.at[idx_v], rows_v, sem).wait()
    pltpu.sync_copy(rows_v, out_hbm.at[pl.ds(base, 16)])


def _sc_gather_rows(table, idx, interpret):
    n, d = idx.shape[0], table.shape[1]
    return pl.kernel(
        _sc_gather_body,
        out_type=jax.ShapeDtypeStruct((n, d), F32),
        mesh=plsc.VectorSubcoreMesh(core_axis_name="c", subcore_axis_name="s"),
        scratch_types=[
            pltpu.VMEM((16,), jnp.int32),
            pltpu.VMEM((16, d), F32),
            pltpu.SemaphoreType.DMA,
        ],
        interpret=interpret,
    )(table, idx)


def _pairwise_d2(q_ref, kt_ref, nq, nk):
    d2 = jnp.zeros((nq, nk), F32)
    for c in range(3):
        diff = q_ref[:, c:c + 1] - kt_ref[c:c + 1, :]
        d2 = d2 + diff * diff
    return d2


def _top3_weights(d2, nq, nk):
    iota = lax.broadcasted_iota(jnp.int32, (nq, nk), 1)
    d = d2
    wt = jnp.zeros((nq, nk), F32)
    den = jnp.zeros((nq, 1), F32)
    for _ in range(3):
        m = jnp.min(d, axis=1, keepdims=True)
        tied = d == m
        jm = jnp.min(jnp.where(tied, iota, nk), axis=1, keepdims=True)
        sel = iota == jm
        w = 1.0 / jnp.maximum(m, 1e-16)
        wt = wt + jnp.where(sel, w, jnp.zeros((), F32))
        den = den + w
        d = jnp.where(sel, jnp.float32(jnp.inf), d)
    return wt, den


def _stage_ab(g_ref, b2row_ref, x2_ref, w31a_ref, w31b_ref,
              b31_ref, w32_ref, b32_ref, pos1_ref, pos2t_ref, b1col_ref,
              x1_ref, w21a_ref, w21b_ref, b21_ref, w22_ref, b22_ref,
              h1_ref):
    h = jnp.dot(g_ref[...], w31a_ref[...], preferred_element_type=F32)
    h = h + jnp.dot(x2_ref[...], w31b_ref[...], preferred_element_type=F32)
    h = jnp.maximum(h + b31_ref[...], 0.0)
    h2 = jnp.dot(h, w32_ref[...], preferred_element_type=F32) + b32_ref[...]

    d2 = _pairwise_d2(pos1_ref, pos2t_ref, 2048, 512)
    mask = (b1col_ref[...] != b2row_ref[...]).astype(F32)
    d2 = d2 + mask * 1e10
    wt, den = _top3_weights(d2, 2048, 512)
    num = jnp.dot(wt, h2, preferred_element_type=F32)
    hi = num / den
    h = jnp.dot(hi, w21a_ref[...], preferred_element_type=F32)
    h = h + jnp.dot(x1_ref[...], w21b_ref[...], preferred_element_type=F32)
    h = jnp.maximum(h + b21_ref[...], 0.0)
    h1_ref[...] = jnp.dot(h, w22_ref[...], preferred_element_type=F32) \
        + b22_ref[...]


def _interp3(pos0, b0col, pos1t_w, b1w, h1_w, bq, nk):
    d2 = jnp.zeros((bq, nk), F32)
    for c in range(3):
        diff = pos0[:, c:c + 1] - pos1t_w[c:c + 1, :]
        d2 = d2 + diff * diff
    mask = (b0col != b1w).astype(F32)
    d2 = d2 + mask * 1e10
    wt, den = _top3_weights(d2, bq, nk)
    num = jnp.dot(wt, h1_w, preferred_element_type=F32)
    return num / den


def _merged_knn_body(g_ref, b2row_ref, x2_ref, w31a_ref, w31b_ref, b31_ref,
                     w32_ref, b32_ref, pos1_ref, pos2t_ref, b1col_ref, x1_ref,
                     w21a_ref, w21b_ref, b21_ref, w22_ref, b22_ref,
                     pos0_ref, b0col_ref, pos1t_ref, b1row_ref,
                     hi_ref, h1_ref, *, bq, win):
    @pl.when(pl.program_id(0) == 0)
    def _ab():
        _stage_ab(g_ref, b2row_ref, x2_ref, w31a_ref, w31b_ref, b31_ref,
                  w32_ref, b32_ref, pos1_ref, pos2t_ref, b1col_ref, x1_ref,
                  w21a_ref, w21b_ref, b21_ref, w22_ref, b22_ref, h1_ref)

    b1row = b1row_ref[...]
    half = bq // 2
    for p in range(2):
        b0col = b0col_ref[p * half:(p + 1) * half, :]
        b_first = jnp.min(b0col)
        b_last = jnp.max(b0col)
        lo = jnp.sum((b1row < b_first).astype(jnp.int32))
        hi_end = jnp.sum((b1row <= b_last).astype(jnp.int32))
        lo_al = jnp.minimum((lo // 128) * 128, 2048 - win)
        fits = (hi_end - lo_al) <= win
        pos0 = pos0_ref[p * half:(p + 1) * half, :]
        osl = pl.ds(p * half, half)

        @pl.when(fits)
        def _window_path(b0col=b0col, pos0=pos0, lo_al=lo_al, osl=osl):
            hi_ref[osl, :] = _interp3(
                pos0, b0col,
                pos1t_ref[:, pl.ds(lo_al, win)],
                b1row_ref[:, pl.ds(lo_al, win)],
                h1_ref[pl.ds(lo_al, win), :], half, win)

        @pl.when(jnp.logical_not(fits))
        def _full_path(b0col=b0col, pos0=pos0, osl=osl):
            hi_ref[osl, :] = _interp3(
                pos0, b0col, pos1t_ref[...], b1row_ref[...],
                h1_ref[...], half, 2048)


def _stage_c_mlp_body(hi_ref, x0_ref, w11a_ref, w11b_ref, b11_ref, w12_ref,
                      b12_ref, w13_ref, b13_ref, wf1_ref, bf1_ref, wf2_ref,
                      bf2_ref, out_ref):
    h = jnp.dot(hi_ref[...], w11a_ref[...], preferred_element_type=F32)
    h = h + jnp.dot(x0_ref[...], w11b_ref[...], preferred_element_type=F32)
    h = jnp.maximum(h + b11_ref[...], 0.0)
    h = jnp.maximum(
        jnp.dot(h, w12_ref[...], preferred_element_type=F32) + b12_ref[...],
        0.0)
    h = jnp.dot(h, w13_ref[...], preferred_element_type=F32) + b13_ref[...]
    h = jnp.maximum(
        jnp.dot(h, wf1_ref[...], preferred_element_type=F32) + bf1_ref[...],
        0.0)
    logit = jnp.dot(h, wf2_ref[...], preferred_element_type=F32) + bf2_ref[...]
    out_ref[...] = jax.nn.sigmoid(logit)


def _run(interpret, x0, pos0, batch0, x1, pos1, batch1, x2, pos2, batch2,
         x3, pos3, batch3, W31, b31, W32, b32, W21, b21, W22, b22,
         W11, b11, W12, b12, W13, b13, Wf1, bf1, Wf2, bf2):
    del pos3, batch3
    b2 = batch2.astype(jnp.int32)
    b1 = batch1.astype(jnp.int32)
    b0 = batch0.astype(jnp.int32)

    g = _sc_gather_rows(x3, b2, interpret)

    bq = 512
    win = 512
    grid = 8192 // bq
    full = lambda shape: pl.BlockSpec(shape, lambda i: (0,) * len(shape))
    blk = lambda shape: pl.BlockSpec(
        shape, lambda i: (i,) + (0,) * (len(shape) - 1))
    hi = pl.pallas_call(
        functools.partial(_merged_knn_body, bq=bq, win=win),
        grid=(grid,),
        in_specs=[
            full((512, 1024)), full((1, 512)), full((512, 256)),
            full((1024, 256)), full((256, 256)), full((1, 256)),
            full((256, 256)), full((1, 256)),
            full((2048, 3)), full((3, 512)), full((2048, 1)),
            full((2048, 128)),
            full((256, 256)), full((128, 256)), full((1, 256)),
            full((256, 128)), full((1, 128)),
            blk((bq, 3)), blk((bq, 1)),
            full((3, 2048)), full((1, 2048)),
        ],
        out_specs=blk((bq, 128)),
        out_shape=jax.ShapeDtypeStruct((8192, 128), F32),
        scratch_shapes=[pltpu.VMEM((2048, 128), F32)],
        interpret=interpret,
    )(
        g, b2[None, :], x2,
        W31[:1024], W31[1024:], b31[None, :], W32, b32[None, :],
        pos1, pos2.T, b1[:, None], x1,
        W21[:256], W21[256:], b21[None, :], W22, b22[None, :],
        pos0, b0[:, None], pos1.T, b1[None, :],
    )

    out = pl.pallas_call(
        _stage_c_mlp_body,
        out_shape=jax.ShapeDtypeStruct((8192, 1), F32),
        interpret=interpret,
    )(
        hi, x0,
        W11[:128], W11[128:], b11[None, :], W12, b12[None, :],
        W13, b13[None, :], Wf1, bf1[None, :], Wf2, bf2[None, :],
    )
    return out


def kernel(x0, pos0, batch0, x1, pos1, batch1, x2, pos2, batch2, x3, pos3,
           batch3, W31, b31, W32, b32, W21, b21, W22, b22, W11, b11, W12, b12,
           W13, b13, Wf1, bf1, Wf2, bf2):
    return _run(False, x0, pos0, batch0, x1, pos1, batch1, x2, pos2, batch2,
                x3, pos3, batch3, W31, b31, W32, b32, W21, b21, W22, b22,
                W11, b11, W12, b12, W13, b13, Wf1, bf1, Wf2, bf2)


# speedup vs baseline: 1.1932x; 1.1932x over previous
import functools

import jax
import jax.numpy as jnp
from jax import lax
from jax.experimental import pallas as pl
from jax.experimental.pallas import tpu as pltpu
from jax.experimental.pallas import tpu_sc as plsc

F32 = jnp.float32

_SC_NC = 2
_SC_NS = 16
_SC_NW = _SC_NC * _SC_NS


def _sc_gather_body(table_hbm, idx_hbm, out_hbm, idx_v, rows_v, sem):
    wid = lax.axis_index("s") * _SC_NC + lax.axis_index("c")
    base = wid * 16
    pltpu.sync_copy(idx_hbm.at[pl.ds(base, 16)], idx_v)
    pltpu.async_copy(table_hbm.at[idx_v], rows_v, sem).wait()
    pltpu.sync_copy(rows_v, out_hbm.at[pl.ds(base, 16)])


def _sc_gather_rows(table, idx, interpret):
    n, d = idx.shape[0], table.shape[1]
    return pl.kernel(
        _sc_gather_body,
        out_type=jax.ShapeDtypeStruct((n, d), F32),
        mesh=plsc.VectorSubcoreMesh(core_axis_name="c", subcore_axis_name="s"),
        scratch_types=[
            pltpu.VMEM((16,), jnp.int32),
            pltpu.VMEM((16, d), F32),
            pltpu.SemaphoreType.DMA,
        ],
        interpret=interpret,
    )(table, idx)


def _pairwise_d2(q_ref, kt_ref, nq, nk):
    d2 = jnp.zeros((nq, nk), F32)
    for c in range(3):
        diff = q_ref[:, c:c + 1] - kt_ref[c:c + 1, :]
        d2 = d2 + diff * diff
    return d2


def _top3_weights(d2, nq, nk):
    iota = lax.broadcasted_iota(jnp.int32, (nq, nk), 1)
    inf = jnp.float32(jnp.inf)
    d = d2
    den = jnp.zeros((nq, 1), F32)
    for _ in range(3):
        m = jnp.min(d, axis=1, keepdims=True)
        tied = d == m
        jm = jnp.min(jnp.where(tied, iota, nk), axis=1, keepdims=True)
        den = den + 1.0 / jnp.maximum(m, 1e-16)
        d = jnp.where(iota == jm, inf, d)
    wt = jnp.where(d == inf, 1.0 / jnp.maximum(d2, 1e-16), jnp.zeros((), F32))
    return wt, den


def _stage_ab(g_ref, b2row_ref, x2_ref, w31a_ref, w31b_ref,
              b31_ref, w32_ref, b32_ref, pos1_ref, pos2t_ref, b1col_ref,
              x1_ref, w21a_ref, w21b_ref, b21_ref, w22_ref, b22_ref,
              h1_ref):
    h = jnp.dot(g_ref[...], w31a_ref[...], preferred_element_type=F32)
    h = h + jnp.dot(x2_ref[...], w31b_ref[...], preferred_element_type=F32)
    h = jnp.maximum(h + b31_ref[...], 0.0)
    h2 = jnp.dot(h, w32_ref[...], preferred_element_type=F32) + b32_ref[...]

    d2 = _pairwise_d2(pos1_ref, pos2t_ref, 2048, 512)
    mask = (b1col_ref[...] != b2row_ref[...]).astype(F32)
    d2 = d2 + mask * 1e10
    wt, den = _top3_weights(d2, 2048, 512)
    num = jnp.dot(wt, h2, preferred_element_type=F32)
    hi = num / den
    h = jnp.dot(hi, w21a_ref[...], preferred_element_type=F32)
    h = h + jnp.dot(x1_ref[...], w21b_ref[...], preferred_element_type=F32)
    h = jnp.maximum(h + b21_ref[...], 0.0)
    h1_ref[...] = jnp.dot(h, w22_ref[...], preferred_element_type=F32) \
        + b22_ref[...]


def _interp3(pos0, b0col, pos1t_w, b1w, h1_w, bq, nk):
    d2 = jnp.zeros((bq, nk), F32)
    for c in range(3):
        diff = pos0[:, c:c + 1] - pos1t_w[c:c + 1, :]
        d2 = d2 + diff * diff
    mask = (b0col != b1w).astype(F32)
    d2 = d2 + mask * 1e10
    wt, den = _top3_weights(d2, bq, nk)
    num = jnp.dot(wt, h1_w, preferred_element_type=F32)
    return num / den


def _merged_knn_body(g_ref, b2row_ref, x2_ref, w31a_ref, w31b_ref, b31_ref,
                     w32_ref, b32_ref, pos1_ref, pos2t_ref, b1col_ref, x1_ref,
                     w21a_ref, w21b_ref, b21_ref, w22_ref, b22_ref,
                     pos0_ref, b0col_ref, pos1t_ref, b1row_ref,
                     hi_ref, h1_ref, *, bq, win):
    @pl.when(pl.program_id(0) == 0)
    def _ab():
        _stage_ab(g_ref, b2row_ref, x2_ref, w31a_ref, w31b_ref, b31_ref,
                  w32_ref, b32_ref, pos1_ref, pos2t_ref, b1col_ref, x1_ref,
                  w21a_ref, w21b_ref, b21_ref, w22_ref, b22_ref, h1_ref)

    b0col = b0col_ref[...]
    b1row = b1row_ref[...]
    b_first = jnp.min(b0col)
    b_last = jnp.max(b0col)
    lo = jnp.sum((b1row < b_first).astype(jnp.int32))
    hi_end = jnp.sum((b1row <= b_last).astype(jnp.int32))
    lo_al = jnp.minimum((lo // 128) * 128, 2048 - win)
    fits = (hi_end - lo_al) <= win

    @pl.when(fits)
    def _window_path():
        hi_ref[...] = _interp3(
            pos0_ref[...], b0col,
            pos1t_ref[:, pl.ds(lo_al, win)], b1row_ref[:, pl.ds(lo_al, win)],
            h1_ref[pl.ds(lo_al, win), :], bq, win)

    @pl.when(jnp.logical_not(fits))
    def _full_path():
        hi_ref[...] = _interp3(
            pos0_ref[...], b0col, pos1t_ref[...], b1row_ref[...],
            h1_ref[...], bq, 2048)


def _stage_c_mlp_body(hi_ref, x0_ref, w11a_ref, w11b_ref, b11_ref, w12_ref,
                      b12_ref, w13_ref, b13_ref, wf1_ref, bf1_ref, wf2_ref,
                      bf2_ref, out_ref):
    h = jnp.dot(hi_ref[...], w11a_ref[...], preferred_element_type=F32)
    h = h + jnp.dot(x0_ref[...], w11b_ref[...], preferred_element_type=F32)
    h = jnp.maximum(h + b11_ref[...], 0.0)
    h = jnp.maximum(
        jnp.dot(h, w12_ref[...], preferred_element_type=F32) + b12_ref[...],
        0.0)
    h = jnp.dot(h, w13_ref[...], preferred_element_type=F32) + b13_ref[...]
    h = jnp.maximum(
        jnp.dot(h, wf1_ref[...], preferred_element_type=F32) + bf1_ref[...],
        0.0)
    logit = jnp.dot(h, wf2_ref[...], preferred_element_type=F32) + bf2_ref[...]
    out_ref[...] = jax.nn.sigmoid(logit)


def _run(interpret, x0, pos0, batch0, x1, pos1, batch1, x2, pos2, batch2,
         x3, pos3, batch3, W31, b31, W32, b32, W21, b21, W22, b22,
         W11, b11, W12, b12, W13, b13, Wf1, bf1, Wf2, bf2):
    del pos3, batch3
    b2 = batch2.astype(jnp.int32)
    b1 = batch1.astype(jnp.int32)
    b0 = batch0.astype(jnp.int32)

    g = _sc_gather_rows(x3, b2, interpret)

    bq = 512
    win = 512
    grid = 8192 // bq
    full = lambda shape: pl.BlockSpec(shape, lambda i: (0,) * len(shape))
    blk = lambda shape: pl.BlockSpec(
        shape, lambda i: (i,) + (0,) * (len(shape) - 1))
    hi = pl.pallas_call(
        functools.partial(_merged_knn_body, bq=bq, win=win),
        grid=(grid,),
        in_specs=[
            full((512, 1024)), full((1, 512)), full((512, 256)),
            full((1024, 256)), full((256, 256)), full((1, 256)),
            full((256, 256)), full((1, 256)),
            full((2048, 3)), full((3, 512)), full((2048, 1)),
            full((2048, 128)),
            full((256, 256)), full((128, 256)), full((1, 256)),
            full((256, 128)), full((1, 128)),
            blk((bq, 3)), blk((bq, 1)),
            full((3, 2048)), full((1, 2048)),
        ],
        out_specs=blk((bq, 128)),
        out_shape=jax.ShapeDtypeStruct((8192, 128), F32),
        scratch_shapes=[pltpu.VMEM((2048, 128), F32)],
        interpret=interpret,
    )(
        g, b2[None, :], x2,
        W31[:1024], W31[1024:], b31[None, :], W32, b32[None, :],
        pos1, pos2.T, b1[:, None], x1,
        W21[:256], W21[256:], b21[None, :], W22, b22[None, :],
        pos0, b0[:, None], pos1.T, b1[None, :],
    )

    out = pl.pallas_call(
        _stage_c_mlp_body,
        out_shape=jax.ShapeDtypeStruct((8192, 1), F32),
        interpret=interpret,
    )(
        hi, x0,
        W11[:128], W11[128:], b11[None, :], W12, b12[None, :],
        W13, b13[None, :], Wf1, bf1[None, :], Wf2, bf2[None, :],
    )
    return out


def kernel(x0, pos0, batch0, x1, pos1, batch1, x2, pos2, batch2, x3, pos3,
           batch3, W31, b31, W32, b32, W21, b21, W22, b22, W11, b11, W12, b12,
           W13, b13, Wf1, bf1, Wf2, bf2):
    return _run(False, x0, pos0, batch0, x1, pos1, batch1, x2, pos2, batch2,
                x3, pos3, batch3, W31, b31, W32, b32, W21, b21, W22, b22,
                W11, b11, W12, b12, W13, b13, Wf1, bf1, Wf2, bf2)
